# baseline (speedup 1.0000x reference)
import functools

import jax
import jax.numpy as jnp
from jax.experimental import pallas as pl
from jax.experimental.pallas import tpu as pltpu


def _vq_block_kernel(x_ref, cb_ref, out_ref, loss_ref, *, inv_n):
    x = x_ref[...]
    cb = cb_ref[...]
    K = cb.shape[0]

    bcol = jnp.sum(cb * cb, axis=1, keepdims=True)
    b = jnp.transpose(bcol, (1, 0))

    a = jnp.sum(x * x, axis=1, keepdims=True)
    c = jax.lax.dot_general(
        x, cb,
        dimension_numbers=(((1,), (1,)), ((), ())),
        preferred_element_type=jnp.float32,
    )
    d = a + b - 2.0 * c

    dmin = jnp.min(d, axis=1, keepdims=True)
    iota = jax.lax.broadcasted_iota(jnp.int32, d.shape, 1)
    idx = jnp.min(jnp.where(d == dmin, iota, K), axis=1, keepdims=True)
    onehot = (iota == idx).astype(jnp.float32)

    q = jax.lax.dot_general(
        onehot, cb,
        dimension_numbers=(((1,), (0,)), ((), ())),
        preferred_element_type=jnp.float32,
    )

    out_ref[...] = q

    part = jnp.sum(dmin).reshape(1, 1)

    @pl.when(pl.program_id(0) == 0)
    def _init():
        loss_ref[...] = jnp.zeros_like(loss_ref)

    loss_ref[...] += part

    @pl.when(pl.program_id(0) == pl.num_programs(0) - 1)
    def _finalize():
        m = loss_ref[...] * inv_n
        loss_ref[...] = m + m


def kernel(x, codebook):
    B, D = x.shape
    K = codebook.shape[0]
    BM = 2048

    out, loss_out = pl.pallas_call(
        functools.partial(_vq_block_kernel, inv_n=1.0 / (B * D)),
        grid=(B // BM,),
        in_specs=[
            pl.BlockSpec((BM, D), lambda i: (i, 0)),
            pl.BlockSpec((K, D), lambda i: (0, 0)),
        ],
        out_specs=[
            pl.BlockSpec((BM, D), lambda i: (i, 0)),
            pl.BlockSpec((1, 1), lambda i: (0, 0)),
        ],
        out_shape=[
            jax.ShapeDtypeStruct((B, D), jnp.float32),
            jax.ShapeDtypeStruct((1, 1), jnp.float32),
        ],
        compiler_params=pltpu.CompilerParams(
            dimension_semantics=("arbitrary",),
        ),
    )(x, codebook)

    return out, loss_out[0, 0]


# speedup vs baseline: 1.2280x; 1.2280x over previous
import functools

import jax
import jax.numpy as jnp
from jax.experimental import pallas as pl
from jax.experimental.pallas import tpu as pltpu


def _vq_block_kernel(x_ref, cb_ref, out_ref, loss_ref, b_ref, *, inv_n):
    K = cb_ref.shape[0]
    cb = cb_ref[...]

    @pl.when(pl.program_id(0) == 0)
    def _prep():
        bcol = jnp.sum(cb * cb, axis=1, keepdims=True)
        b_ref[0:1, 0:K] = jnp.transpose(bcol, (1, 0))

    x = x_ref[...]
    b = b_ref[0:1, 0:K]

    a = jnp.sum(x * x, axis=1, keepdims=True)
    c = jax.lax.dot_general(
        x, cb,
        dimension_numbers=(((1,), (1,)), ((), ())),
        preferred_element_type=jnp.float32,
    )
    d = a + b - 2.0 * c

    dmin = jnp.min(d, axis=1, keepdims=True)
    iota = jax.lax.broadcasted_iota(jnp.int32, d.shape, 1).astype(jnp.float32)
    idx = jnp.min(jnp.where(d == dmin, iota, jnp.float32(K)),
                  axis=1, keepdims=True)
    onehot = (iota == idx).astype(jnp.float32)

    q = jax.lax.dot_general(
        onehot, cb,
        dimension_numbers=(((1,), (0,)), ((), ())),
        preferred_element_type=jnp.float32,
    )

    out_ref[...] = q

    part = jnp.sum(dmin).reshape(1, 1)

    @pl.when(pl.program_id(0) == 0)
    def _init():
        loss_ref[...] = jnp.zeros_like(loss_ref)

    loss_ref[...] += part

    @pl.when(pl.program_id(0) == pl.num_programs(0) - 1)
    def _finalize():
        m = loss_ref[...] * inv_n
        loss_ref[...] = m + m


def kernel(x, codebook):
    B, D = x.shape
    K = codebook.shape[0]
    BM = 4096

    out, loss_out = pl.pallas_call(
        functools.partial(_vq_block_kernel, inv_n=1.0 / (B * D)),
        grid=(B // BM,),
        in_specs=[
            pl.BlockSpec((BM, D), lambda i: (i, 0)),
            pl.BlockSpec((K, D), lambda i: (0, 0)),
        ],
        out_specs=[
            pl.BlockSpec((BM, D), lambda i: (i, 0)),
            pl.BlockSpec((1, 1), lambda i: (0, 0)),
        ],
        out_shape=[
            jax.ShapeDtypeStruct((B, D), jnp.float32),
            jax.ShapeDtypeStruct((1, 1), jnp.float32),
        ],
        scratch_shapes=[pltpu.VMEM((1, 128), jnp.float32)],
        compiler_params=pltpu.CompilerParams(
            dimension_semantics=("arbitrary",),
        ),
    )(x, codebook)

    return out, loss_out[0, 0]
